# Initial kernel scaffold; baseline (speedup 1.0000x reference)
import functools

import jax
import jax.numpy as jnp
from jax import lax
from jax.experimental import pallas as pl
from jax.experimental.pallas import tpu as pltpu
from jax.experimental.pallas import tpu_sc as plsc

NC = 2
NS = 16
LANE = 16
EPS = 1e-6


def _dot(x, w):
    return jnp.dot(x, w, preferred_element_type=jnp.float32)


def _tc_pre0_body(h_ref, eemb_ref, wa, ba, wb, bb, wd, bd, we, be, wc, bc,
                  ha_o, hb_o, hd_o, he_o, cet_o):
    x = h_ref[...]
    ha_o[...] = _dot(x, wa[...]) + ba[...]
    hb_o[...] = _dot(x, wb[...]) + bb[...]
    hd_o[...] = _dot(x, wd[...]) + bd[...]
    he_o[...] = _dot(x, we[...]) + be[...]

    @pl.when(pl.program_id(0) == 0)
    def _():
        cet_o[...] = _dot(eemb_ref[...], wc[...]) + bc[...]


def _tc_step_body(h_ref, hA_ref, num_ref, den_ref,
                  wa, ba, wb, bb, wd, bd, we, be,
                  h_o, ha_o, hb_o, hd_o, he_o):
    num = jnp.concatenate([num_ref[0], num_ref[1]], axis=-1)
    den = jnp.concatenate([den_ref[0], den_ref[1]], axis=-1)
    h = h_ref[...] + jnp.maximum(hA_ref[...] + num / (den + EPS), 0.0)
    h_o[...] = h
    ha_o[...] = _dot(h, wa[...]) + ba[...]
    hb_o[...] = _dot(h, wb[...]) + bb[...]
    hd_o[...] = _dot(h, wd[...]) + bd[...]
    he_o[...] = _dot(h, we[...]) + be[...]


def _tc_fin_body(h_ref, hA_ref, num_ref, den_ref, h_o):
    num = jnp.concatenate([num_ref[0], num_ref[1]], axis=-1)
    den = jnp.concatenate([den_ref[0], den_ref[1]], axis=-1)
    h_o[...] = h_ref[...] + jnp.maximum(hA_ref[...] + num / (den + EPS), 0.0)


def _tc_ce_body(e_ref, wc, bc, o_ref):
    x = jnp.concatenate([e_ref[0], e_ref[1]], axis=-1)
    y = _dot(x, wc[...]) + bc[...]
    h2 = y.shape[-1] // 2
    o_ref[0] = y[:, :h2]
    o_ref[1] = y[:, h2:]


def _full(block):
    return pl.BlockSpec(block, lambda i: tuple(0 for _ in block))


def _rows(bn, w):
    return pl.BlockSpec((bn, w), lambda i: (i, 0))


def _rows3(bn, w):
    return pl.BlockSpec((NC, bn, w), lambda i: (0, i, 0))


def _make_sc_edge(mode_table, write_e, n_pad, E, hid, eb):
    h2 = hid // 2
    per_tile = E // NS
    n_blocks = per_tile // eb
    rows_per_tile = n_pad // NS
    rc = eb
    n_chunks = rows_per_tile // rc

    mesh = plsc.VectorSubcoreMesh(core_axis_name="c", subcore_axis_name="s",
                                  num_cores=NC, num_subcores=NS)

    out_type = []
    if write_e:
        out_type.append(jax.ShapeDtypeStruct((NC * E, h2), jnp.float32))
    out_type.append(jax.ShapeDtypeStruct((NC * n_pad, h2), jnp.float32))
    out_type.append(jax.ShapeDtypeStruct((NC * n_pad, h2), jnp.float32))

    scratch = [
        pltpu.VMEM((eb,), jnp.int32),
        pltpu.VMEM((eb,), jnp.int32),
        pltpu.VMEM((eb,), jnp.int32),
        pltpu.VMEM((eb,), jnp.int32),
        pltpu.VMEM((eb,), jnp.int32),
        pltpu.VMEM((eb, h2), jnp.float32),
        pltpu.VMEM((eb, h2), jnp.float32),
        pltpu.VMEM((eb, h2), jnp.float32),
        pltpu.VMEM((eb, h2), jnp.float32),
        pltpu.VMEM((eb, h2), jnp.float32),
        pltpu.VMEM((eb, h2), jnp.float32),
        pltpu.VMEM((eb, h2), jnp.float32),
        pltpu.VMEM((eb, h2), jnp.float32),
        pltpu.VMEM((rc, h2), jnp.float32),
        pltpu.VMEM_SHARED((n_pad, h2), jnp.float32),
        pltpu.VMEM_SHARED((n_pad, h2), jnp.float32),
        pltpu.SemaphoreType.DMA,
    ]

    def body(*refs):
        it = iter(refs)
        src_h = next(it)
        dst_h = next(it)
        if mode_table:
            et_h = next(it)
        dh2 = next(it)
        eh2 = next(it)
        bh2 = next(it)
        if mode_table:
            cet2 = next(it)
            eintab2 = next(it)
        else:
            ce2 = next(it)
            if write_e:
                ein2 = next(it)
        if write_e:
            eout = next(it)
        num_out = next(it)
        den_out = next(it)
        (srcb, dstb, isrc2, idst2, ietb, gD, gE, gB, ceb, einb, sbuf,
         contrib, enew, zbuf, num_acc, den_acc, sem) = it

        c = lax.axis_index("c")
        s = lax.axis_index("s")

        zero16 = jnp.zeros((LANE,), jnp.float32)

        def zrow(i, carry):
            for j in range(h2 // LANE):
                zbuf[i, pl.ds(j * LANE, LANE)] = zero16
            return carry

        lax.fori_loop(0, rc, zrow, 0)
        for k in range(n_chunks):
            r0 = s * rows_per_tile + k * rc
            pltpu.sync_copy(zbuf, num_acc.at[pl.ds(r0, rc)])
            pltpu.sync_copy(zbuf, den_acc.at[pl.ds(r0, rc)])
        plsc.subcore_barrier()

        tile_e0 = s * per_tile

        def blk(b, carry):
            base = tile_e0 + b * eb
            pltpu.sync_copy(src_h.at[pl.ds(base, eb)], srcb)
            pltpu.sync_copy(dst_h.at[pl.ds(base, eb)], dstb)
            if mode_table:
                pltpu.sync_copy(et_h.at[pl.ds(base, eb)], ietb)
            for k in range(eb // LANE):
                sl = pl.ds(k * LANE, LANE)
                isrc2[sl] = srcb[sl] * 2 + c
                idst2[sl] = dstb[sl] * 2 + c
                if mode_table:
                    ietb[sl] = ietb[sl] * 2 + c
            cps = [pltpu.async_copy(dh2.at[isrc2], gD, sem),
                   pltpu.async_copy(eh2.at[idst2], gE, sem),
                   pltpu.async_copy(bh2.at[isrc2], gB, sem)]
            if mode_table:
                cps.append(pltpu.async_copy(cet2.at[ietb], ceb, sem))
                cps.append(pltpu.async_copy(eintab2.at[ietb], einb, sem))
            else:
                row0 = c * E + base
                cps.append(pltpu.async_copy(ce2.at[pl.ds(row0, eb)], ceb, sem))
                if write_e:
                    cps.append(pltpu.async_copy(ein2.at[pl.ds(row0, eb)],
                                                einb, sem))
            for cp in cps:
                cp.wait()

            def row(i, carry2):
                for j in range(h2 // LANE):
                    sl = pl.ds(j * LANE, LANE)
                    x = gD[i, sl] + gE[i, sl] + ceb[i, sl]
                    sg = 1.0 / (1.0 + jnp.exp(-x))
                    sbuf[i, sl] = sg
                    contrib[i, sl] = sg * gB[i, sl]
                    if write_e:
                        enew[i, sl] = einb[i, sl] + jnp.maximum(x, 0.0)
                return carry2

            lax.fori_loop(0, eb, row, 0)

            pltpu.sync_copy(contrib, num_acc.at[dstb], add=True)
            pltpu.sync_copy(sbuf, den_acc.at[dstb], add=True)
            if write_e:
                pltpu.sync_copy(enew, eout.at[pl.ds(c * E + base, eb)])
            return carry

        lax.fori_loop(0, n_blocks, blk, 0)
        plsc.subcore_barrier()

        for k in range(n_chunks):
            r0 = s * rows_per_tile + k * rc
            pltpu.sync_copy(num_acc.at[pl.ds(r0, rc)], zbuf)
            pltpu.sync_copy(zbuf, num_out.at[pl.ds(c * n_pad + r0, rc)])
            pltpu.sync_copy(den_acc.at[pl.ds(r0, rc)], zbuf)
            pltpu.sync_copy(zbuf, den_out.at[pl.ds(c * n_pad + r0, rc)])

    return pl.kernel(body, out_type=tuple(out_type), mesh=mesh,
                     scratch_types=scratch)


def kernel(node_id, edge_index, edge_type, h_emb, e_emb,
           A_w, A_b, B_w, B_b, C_w, C_b, D_w, D_b, E_w, E_b):
    N, hid = h_emb.shape
    E = edge_index.shape[1]
    L = A_w.shape[0]
    h2 = hid // 2
    eb = 80
    n_pad = ((N + NS * eb - 1) // (NS * eb)) * (NS * eb)
    bn = 1000
    be = 2000

    src = edge_index[0]
    dst = edge_index[1]

    w128 = _full((hid, hid))
    b128 = _full((1, hid))

    A_b2, B_b2, C_b2, D_b2, E_b2 = (b.reshape(1, hid)
                                    for b in (A_b, B_b, C_b, D_b, E_b))

    nsd = jax.ShapeDtypeStruct((N, hid), jnp.float32)

    tc_pre0 = pl.pallas_call(
        _tc_pre0_body,
        grid=(N // bn,),
        in_specs=[_rows(bn, hid), _full(e_emb.shape)] + [w128, b128] * 5,
        out_specs=[_rows(bn, hid)] * 4 + [_full(e_emb.shape)],
        out_shape=[nsd] * 4 + [jax.ShapeDtypeStruct(e_emb.shape, jnp.float32)],
    )

    tc_step = pl.pallas_call(
        _tc_step_body,
        grid=(N // bn,),
        in_specs=[_rows(bn, hid), _rows(bn, hid), _rows3(bn, h2),
                  _rows3(bn, h2)] + [w128, b128] * 4,
        out_specs=[_rows(bn, hid)] * 5,
        out_shape=[nsd] * 5,
    )

    tc_fin = pl.pallas_call(
        _tc_fin_body,
        grid=(N // bn,),
        in_specs=[_rows(bn, hid), _rows(bn, hid), _rows3(bn, h2),
                  _rows3(bn, h2)],
        out_specs=_rows(bn, hid),
        out_shape=nsd,
    )

    tc_ce = pl.pallas_call(
        _tc_ce_body,
        grid=(E // be,),
        in_specs=[_rows3(be, h2), w128, b128],
        out_specs=_rows3(be, h2),
        out_shape=jax.ShapeDtypeStruct((NC, E, h2), jnp.float32),
    )

    sc_edge0 = _make_sc_edge(True, True, n_pad, E, hid, eb)
    sc_edge_mid = _make_sc_edge(False, True, n_pad, E, hid, eb)
    sc_edge_last = _make_sc_edge(False, False, n_pad, E, hid, eb)

    def v2(a):
        return a.reshape(-1, h2)

    def v3(a):
        return a.reshape(NC, -1, h2)

    h = h_emb

    Ah, Bh, Dh, Eh, CeT = tc_pre0(h, e_emb, A_w[0], A_b2, B_w[0], B_b2,
                                  D_w[0], D_b2, E_w[0], E_b2, C_w[0], C_b2)
    e_state, num, den = sc_edge0(src, dst, edge_type, v2(Dh), v2(Eh), v2(Bh),
                                 v2(CeT), v2(e_emb))

    for l in range(1, L):
        h, Ah, Bh, Dh, Eh = tc_step(h, Ah, v3(num), v3(den),
                                    A_w[l], A_b2, B_w[l], B_b2,
                                    D_w[l], D_b2, E_w[l], E_b2)
        Ce = tc_ce(v3(e_state), C_w[l], C_b2)
        if l < L - 1:
            e_state, num, den = sc_edge_mid(src, dst, v2(Dh), v2(Eh), v2(Bh),
                                            Ce.reshape(-1, h2), e_state)
        else:
            num, den = sc_edge_last(src, dst, v2(Dh), v2(Eh), v2(Bh),
                                    Ce.reshape(-1, h2))

    return tc_fin(h, Ah, v3(num), v3(den))


# speedup vs baseline: 2.0215x; 2.0215x over previous
import jax
import jax.numpy as jnp
from jax import lax
from jax.experimental import pallas as pl
from jax.experimental.pallas import tpu as pltpu
from jax.experimental.pallas import tpu_sc as plsc

NC = 2
NS = 16
LANE = 16
EPS = 1e-6


def _dot(x, w):
    return jnp.dot(x, w, preferred_element_type=jnp.float32)


def _tc_pre0_body(h_ref, eemb_ref, wa, ba, wb, bb, wd, bd, we, be, wc, bc,
                  ha_o, hb_o, hd_o, he_o, cet_o):
    x = h_ref[...]
    ha_o[...] = _dot(x, wa[...]) + ba[...]
    hb_o[...] = _dot(x, wb[...]) + bb[...]
    hd_o[...] = _dot(x, wd[...]) + bd[...]
    he_o[...] = _dot(x, we[...]) + be[...]

    @pl.when(pl.program_id(0) == 0)
    def _():
        cet_o[...] = _dot(eemb_ref[...], wc[...]) + bc[...]


def _tc_step_body(h_ref, hA_ref, num_ref, den_ref,
                  wa, ba, wb, bb, wd, bd, we, be,
                  h_o, ha_o, hb_o, hd_o, he_o):
    h = h_ref[...] + jnp.maximum(
        hA_ref[...] + num_ref[...] / (den_ref[...] + EPS), 0.0)
    h_o[...] = h
    ha_o[...] = _dot(h, wa[...]) + ba[...]
    hb_o[...] = _dot(h, wb[...]) + bb[...]
    hd_o[...] = _dot(h, wd[...]) + bd[...]
    he_o[...] = _dot(h, we[...]) + be[...]


def _tc_fin_body(h_ref, hA_ref, num_ref, den_ref, h_o):
    h_o[...] = h_ref[...] + jnp.maximum(
        hA_ref[...] + num_ref[...] / (den_ref[...] + EPS), 0.0)


def _tc_ce_body(e_ref, wc, bc, o_ref):
    o_ref[...] = _dot(e_ref[...], wc[...]) + bc[...]


def _full(block):
    return pl.BlockSpec(block, lambda i: tuple(0 for _ in block))


def _rows(bn, w):
    return pl.BlockSpec((bn, w), lambda i: (i, 0))


def _make_sc_edge(mode_table, write_e, n_pad, E, hid, eb):
    per_tile = E // NS
    n_blocks = per_tile // eb
    rows_per_tile = n_pad // NS
    rc = eb
    n_chunks = rows_per_tile // rc

    mesh = plsc.VectorSubcoreMesh(core_axis_name="c", subcore_axis_name="s",
                                  num_cores=NC, num_subcores=NS)

    out_type = []
    if write_e:
        out_type.append(jax.ShapeDtypeStruct((E, hid), jnp.float32))
    out_type.append(jax.ShapeDtypeStruct((n_pad, hid), jnp.float32))
    out_type.append(jax.ShapeDtypeStruct((n_pad, hid), jnp.float32))

    scratch = [
        pltpu.VMEM((eb,), jnp.int32),
        pltpu.VMEM((eb,), jnp.int32),
        pltpu.VMEM((eb,), jnp.int32),
        pltpu.VMEM((eb, hid), jnp.float32),
        pltpu.VMEM((eb, hid), jnp.float32),
        pltpu.VMEM((eb, hid), jnp.float32),
        pltpu.VMEM((eb, hid), jnp.float32),
        pltpu.VMEM_SHARED((n_pad, hid), jnp.float32),
        pltpu.SemaphoreType.DMA,
    ]

    def body(*refs):
        it = iter(refs)
        src_h = next(it)
        dst_h = next(it)
        if mode_table:
            et_h = next(it)
        dh = next(it)
        eh = next(it)
        bh = next(it)
        if mode_table:
            cet = next(it)
            eintab = next(it)
        else:
            ce2 = next(it)
            if write_e:
                ein2 = next(it)
        if write_e:
            eout = next(it)
        num_out = next(it)
        den_out = next(it)
        (srcb, dstb, ietb, gD, gE, aux, ceb, acc, sem) = it

        c = lax.axis_index("c")
        s = lax.axis_index("s")
        is_num = c == 0

        zero16 = jnp.zeros((LANE,), jnp.float32)

        def zrow(i, carry):
            for j in range(hid // LANE):
                ceb[i, pl.ds(j * LANE, LANE)] = zero16
            return carry

        lax.fori_loop(0, rc, zrow, 0)
        for k in range(n_chunks):
            r0 = s * rows_per_tile + k * rc
            pltpu.sync_copy(ceb, acc.at[pl.ds(r0, rc)])
        plsc.subcore_barrier()

        tile_e0 = s * per_tile

        def blk(b, carry):
            base = tile_e0 + b * eb
            pltpu.sync_copy(src_h.at[pl.ds(base, eb)], srcb)
            pltpu.sync_copy(dst_h.at[pl.ds(base, eb)], dstb)
            if mode_table:
                pltpu.sync_copy(et_h.at[pl.ds(base, eb)], ietb)
            cps = [pltpu.async_copy(dh.at[srcb], gD, sem),
                   pltpu.async_copy(eh.at[dstb], gE, sem)]
            if mode_table:
                cps.append(pltpu.async_copy(cet.at[ietb], ceb, sem))
            else:
                cps.append(pltpu.async_copy(ce2.at[pl.ds(base, eb)], ceb, sem))
            for cp in cps:
                cp.wait()

            @pl.when(is_num)
            def _():
                pltpu.async_copy(bh.at[srcb], aux, sem).wait()

            if write_e:
                @pl.when(~is_num)
                def _():
                    if mode_table:
                        pltpu.async_copy(eintab.at[ietb], aux, sem).wait()
                    else:
                        pltpu.async_copy(ein2.at[pl.ds(base, eb)], aux,
                                         sem).wait()

            def row_num(i, carry2):
                for j in range(hid // LANE):
                    sl = pl.ds(j * LANE, LANE)
                    x = gD[i, sl] + gE[i, sl] + ceb[i, sl]
                    sg = 1.0 / (1.0 + jnp.exp(-x))
                    gD[i, sl] = sg * aux[i, sl]
                return carry2

            def row_den(i, carry2):
                for j in range(hid // LANE):
                    sl = pl.ds(j * LANE, LANE)
                    x = gD[i, sl] + gE[i, sl] + ceb[i, sl]
                    sg = 1.0 / (1.0 + jnp.exp(-x))
                    gD[i, sl] = sg
                    if write_e:
                        aux[i, sl] = aux[i, sl] + jnp.maximum(x, 0.0)
                return carry2

            @pl.when(is_num)
            def _():
                lax.fori_loop(0, eb, row_num, 0)

            @pl.when(~is_num)
            def _():
                lax.fori_loop(0, eb, row_den, 0)

            pltpu.sync_copy(gD, acc.at[dstb], add=True)
            if write_e:
                @pl.when(~is_num)
                def _():
                    pltpu.sync_copy(aux, eout.at[pl.ds(base, eb)])
            return carry

        lax.fori_loop(0, n_blocks, blk, 0)
        plsc.subcore_barrier()

        for k in range(n_chunks):
            r0 = s * rows_per_tile + k * rc
            pltpu.sync_copy(acc.at[pl.ds(r0, rc)], ceb)

            @pl.when(is_num)
            def _():
                pltpu.sync_copy(ceb, num_out.at[pl.ds(r0, rc)])

            @pl.when(~is_num)
            def _():
                pltpu.sync_copy(ceb, den_out.at[pl.ds(r0, rc)])

    return pl.kernel(body, out_type=tuple(out_type), mesh=mesh,
                     scratch_types=scratch)


def kernel(node_id, edge_index, edge_type, h_emb, e_emb,
           A_w, A_b, B_w, B_b, C_w, C_b, D_w, D_b, E_w, E_b):
    N, hid = h_emb.shape
    E = edge_index.shape[1]
    L = A_w.shape[0]
    eb = 80
    n_pad = ((N + NS * eb - 1) // (NS * eb)) * (NS * eb)
    bn = 1000
    be = 2000

    src = edge_index[0]
    dst = edge_index[1]

    w128 = _full((hid, hid))
    b128 = _full((1, hid))

    def bias(b, l):
        return b[l].reshape(1, hid)

    nsd = jax.ShapeDtypeStruct((N, hid), jnp.float32)

    tc_pre0 = pl.pallas_call(
        _tc_pre0_body,
        grid=(N // bn,),
        in_specs=[_rows(bn, hid), _full(e_emb.shape)] + [w128, b128] * 5,
        out_specs=[_rows(bn, hid)] * 4 + [_full(e_emb.shape)],
        out_shape=[nsd] * 4 + [jax.ShapeDtypeStruct(e_emb.shape, jnp.float32)],
    )

    tc_step = pl.pallas_call(
        _tc_step_body,
        grid=(N // bn,),
        in_specs=[_rows(bn, hid)] * 4 + [w128, b128] * 4,
        out_specs=[_rows(bn, hid)] * 5,
        out_shape=[nsd] * 5,
    )

    tc_fin = pl.pallas_call(
        _tc_fin_body,
        grid=(N // bn,),
        in_specs=[_rows(bn, hid)] * 4,
        out_specs=_rows(bn, hid),
        out_shape=nsd,
    )

    tc_ce = pl.pallas_call(
        _tc_ce_body,
        grid=(E // be,),
        in_specs=[_rows(be, hid), w128, b128],
        out_specs=_rows(be, hid),
        out_shape=jax.ShapeDtypeStruct((E, hid), jnp.float32),
    )

    sc_edge0 = _make_sc_edge(True, True, n_pad, E, hid, eb)
    sc_edge_mid = _make_sc_edge(False, True, n_pad, E, hid, eb)
    sc_edge_last = _make_sc_edge(False, False, n_pad, E, hid, eb)

    h = h_emb

    Ah, Bh, Dh, Eh, CeT = tc_pre0(h, e_emb,
                                  A_w[0], bias(A_b, 0), B_w[0], bias(B_b, 0),
                                  D_w[0], bias(D_b, 0), E_w[0], bias(E_b, 0),
                                  C_w[0], bias(C_b, 0))
    e_state, num, den = sc_edge0(src, dst, edge_type, Dh, Eh, Bh, CeT, e_emb)

    for l in range(1, L):
        h, Ah, Bh, Dh, Eh = tc_step(h, Ah, num, den,
                                    A_w[l], bias(A_b, l), B_w[l], bias(B_b, l),
                                    D_w[l], bias(D_b, l), E_w[l], bias(E_b, l))
        Ce = tc_ce(e_state, C_w[l], bias(C_b, l))
        if l < L - 1:
            e_state, num, den = sc_edge_mid(src, dst, Dh, Eh, Bh, Ce, e_state)
        else:
            num, den = sc_edge_last(src, dst, Dh, Eh, Bh, Ce)

    return tc_fin(h, Ah, num, den)


# speedup vs baseline: 2.3187x; 1.1470x over previous
import jax
import jax.numpy as jnp
from jax import lax
from jax.experimental import pallas as pl
from jax.experimental.pallas import tpu as pltpu
from jax.experimental.pallas import tpu_sc as plsc

NC = 2
NS = 16
LANE = 16
EPS = 1e-6


def _dot(x, w):
    return jnp.dot(x, w, preferred_element_type=jnp.float32)


def _tc_pre0_body(h_ref, eemb_ref, wa, ba, wb, bb, wd, bd, we, be, wc, bc,
                  ha_o, hb_o, hd_o, he_o, cet_o, eem_o):
    x = h_ref[...]
    ha_o[...] = _dot(x, wa[...]) + ba[...]
    hb_o[...] = _dot(x, wb[...]) + bb[...]
    hd_o[...] = _dot(x, wd[...]) + bd[...]
    he_o[...] = _dot(x, we[...]) + be[...]

    @pl.when(pl.program_id(0) == 0)
    def _():
        em = eemb_ref[...]
        ce = _dot(em, wc[...]) + bc[...]
        n = em.shape[0]
        for k in range(NC * NS):
            cet_o[pl.ds(k * n, n), :] = ce
            eem_o[pl.ds(k * n, n), :] = em


def _tc_step_body(h_ref, hA_ref, num_ref, den_ref,
                  wa, ba, wb, bb, wd, bd, we, be,
                  h_o, ha_o, hb_o, hd_o, he_o):
    h = h_ref[...] + jnp.maximum(
        hA_ref[...] + num_ref[...] / (den_ref[...] + EPS), 0.0)
    h_o[...] = h
    ha_o[...] = _dot(h, wa[...]) + ba[...]
    hb_o[...] = _dot(h, wb[...]) + bb[...]
    hd_o[...] = _dot(h, wd[...]) + bd[...]
    he_o[...] = _dot(h, we[...]) + be[...]


def _tc_fin_body(h_ref, hA_ref, num_ref, den_ref, h_o):
    h_o[...] = h_ref[...] + jnp.maximum(
        hA_ref[...] + num_ref[...] / (den_ref[...] + EPS), 0.0)


def _tc_ce_body(e_ref, wc, bc, o_ref):
    o_ref[...] = _dot(e_ref[...], wc[...]) + bc[...]


def _full(block):
    return pl.BlockSpec(block, lambda i: tuple(0 for _ in block))


def _rows(bn, w):
    return pl.BlockSpec((bn, w), lambda i: (i, 0))


def _make_sc_edge(mode_table, write_e, n_pad, E, hid, eb):
    per_tile = E // NS
    n_blocks = per_tile // eb
    rows_per_tile = n_pad // NS
    rc = eb
    n_chunks = rows_per_tile // rc

    mesh = plsc.VectorSubcoreMesh(core_axis_name="c", subcore_axis_name="s",
                                  num_cores=NC, num_subcores=NS)

    out_type = []
    if write_e:
        out_type.append(jax.ShapeDtypeStruct((E, hid), jnp.float32))
    out_type.append(jax.ShapeDtypeStruct((n_pad, hid), jnp.float32))
    out_type.append(jax.ShapeDtypeStruct((n_pad, hid), jnp.float32))

    hb = eb // 2

    scratch = [
        pltpu.VMEM((hb,), jnp.int32),
        pltpu.VMEM((hb,), jnp.int32),
        pltpu.VMEM((hb,), jnp.int32),
        pltpu.VMEM((hb,), jnp.int32),
        pltpu.VMEM((eb,), jnp.int32),
        pltpu.VMEM((eb, hid), jnp.float32),
        pltpu.VMEM((eb, hid), jnp.float32),
        pltpu.VMEM((eb, hid), jnp.float32),
        pltpu.VMEM((eb, hid), jnp.float32),
        pltpu.VMEM_SHARED((n_pad, hid), jnp.float32),
        pltpu.SemaphoreType.DMA,
        pltpu.SemaphoreType.DMA,
        pltpu.SemaphoreType.DMA,
    ]

    def body(*refs):
        it = iter(refs)
        src_h = next(it)
        dst_h = next(it)
        if mode_table:
            et_h = next(it)
        dh = next(it)
        eh = next(it)
        bh = next(it)
        if mode_table:
            cet = next(it)
            eintab = next(it)
        else:
            ce2 = next(it)
            if write_e:
                ein2 = next(it)
        if write_e:
            eout = next(it)
        num_out = next(it)
        den_out = next(it)
        (srcb0, srcb1, dstb0, dstb1, ietb, gD, gE, aux, ceb, acc,
         sem_a, sem_b, sem_e) = it

        c = lax.axis_index("c")
        s = lax.axis_index("s")
        is_num = c == 0

        zero16 = jnp.zeros((LANE,), jnp.float32)

        def zrow(i, carry):
            for j in range(hid // LANE):
                ceb[i, pl.ds(j * LANE, LANE)] = zero16
            return carry

        lax.fori_loop(0, rc, zrow, 0)
        for k in range(n_chunks):
            r0 = s * rows_per_tile + k * rc
            pltpu.sync_copy(ceb, acc.at[pl.ds(r0, rc)])
        plsc.subcore_barrier()

        tile_e0 = s * per_tile

        def half(x):
            return x.at[pl.ds(0, hb)], x.at[pl.ds(hb, hb)]

        gD0, gD1 = half(gD)
        gE0, gE1 = half(gE)
        aux0, aux1 = half(aux)

        def wait_rows(dst, sem):
            pltpu.make_async_copy(dh.at[pl.ds(0, dst.shape[0])], dst,
                                  sem).wait()

        def blk(b, carry):
            base = tile_e0 + b * eb

            if write_e:
                @pl.when((~is_num) & (b > 0))
                def _():
                    pltpu.make_async_copy(dh.at[pl.ds(0, eb)], aux,
                                          sem_e).wait()

            pltpu.sync_copy(src_h.at[pl.ds(base, hb)], srcb0)
            pltpu.sync_copy(src_h.at[pl.ds(base + hb, hb)], srcb1)
            pltpu.sync_copy(dst_h.at[pl.ds(base, hb)], dstb0)
            pltpu.sync_copy(dst_h.at[pl.ds(base + hb, hb)], dstb1)
            if mode_table:
                pltpu.sync_copy(et_h.at[pl.ds(base, eb)], ietb)
                rep = (c * NS + s) * 16
                for k in range(eb // LANE):
                    sl = pl.ds(k * LANE, LANE)
                    ietb[sl] = ietb[sl] + rep

            if mode_table:
                pltpu.async_copy(cet.at[ietb], ceb, sem_a)
            else:
                pltpu.async_copy(ce2.at[pl.ds(base, eb)], ceb, sem_a)

            if write_e and mode_table:
                @pl.when(~is_num)
                def _():
                    pltpu.async_copy(eintab.at[ietb], aux, sem_a)

            pltpu.async_copy(dh.at[srcb0], gD0, sem_a)
            pltpu.async_copy(eh.at[dstb0], gE0, sem_a)
            pltpu.async_copy(dh.at[srcb1], gD1, sem_b)
            pltpu.async_copy(eh.at[dstb1], gE1, sem_b)

            @pl.when(is_num)
            def _():
                pltpu.async_copy(bh.at[srcb0], aux0, sem_a)
                pltpu.async_copy(bh.at[srcb1], aux1, sem_b)

            if write_e and not mode_table:
                @pl.when(~is_num)
                def _():
                    pltpu.async_copy(ein2.at[pl.ds(base, hb)], aux0, sem_a)
                    pltpu.async_copy(ein2.at[pl.ds(base + hb, hb)], aux1,
                                     sem_b)

            def make_row_num(off):
                def row_num(i, carry2):
                    for j in range(hid // LANE):
                        sl = pl.ds(j * LANE, LANE)
                        x = gD[i + off, sl] + gE[i + off, sl] + ceb[i + off,
                                                                    sl]
                        sg = 1.0 / (1.0 + jnp.exp(-x))
                        gD[i + off, sl] = sg * aux[i + off, sl]
                    return carry2
                return row_num

            def make_row_den(off):
                def row_den(i, carry2):
                    for j in range(hid // LANE):
                        sl = pl.ds(j * LANE, LANE)
                        x = gD[i + off, sl] + gE[i + off, sl] + ceb[i + off,
                                                                    sl]
                        sg = 1.0 / (1.0 + jnp.exp(-x))
                        gD[i + off, sl] = sg
                        if write_e:
                            aux[i + off, sl] = (aux[i + off, sl]
                                                + jnp.maximum(x, 0.0))
                    return carry2
                return row_den

            def compute_half(idx):
                off = idx * hb

                @pl.when(is_num)
                def _():
                    lax.fori_loop(0, hb, make_row_num(off), 0)

                @pl.when(~is_num)
                def _():
                    lax.fori_loop(0, hb, make_row_den(off), 0)

            wait_rows(ceb, sem_a)
            wait_rows(gD0, sem_a)
            wait_rows(gE0, sem_a)

            @pl.when(is_num)
            def _():
                wait_rows(aux0, sem_a)

            if write_e:
                @pl.when(~is_num)
                def _():
                    if mode_table:
                        wait_rows(aux, sem_a)
                    else:
                        wait_rows(aux0, sem_a)

            compute_half(0)
            pltpu.sync_copy(gD0, acc.at[dstb0], add=True)

            wait_rows(gD1, sem_b)
            wait_rows(gE1, sem_b)

            @pl.when(is_num)
            def _():
                wait_rows(aux1, sem_b)

            if write_e and not mode_table:
                @pl.when(~is_num)
                def _():
                    wait_rows(aux1, sem_b)

            compute_half(1)
            pltpu.sync_copy(gD1, acc.at[dstb1], add=True)

            if write_e:
                @pl.when(~is_num)
                def _():
                    pltpu.async_copy(aux, eout.at[pl.ds(base, eb)], sem_e)
            return carry

        lax.fori_loop(0, n_blocks, blk, 0)
        if write_e:
            @pl.when(~is_num)
            def _():
                pltpu.make_async_copy(dh.at[pl.ds(0, eb)], aux, sem_e).wait()
        plsc.subcore_barrier()

        for k in range(n_chunks):
            r0 = s * rows_per_tile + k * rc
            pltpu.sync_copy(acc.at[pl.ds(r0, rc)], ceb)

            @pl.when(is_num)
            def _():
                pltpu.sync_copy(ceb, num_out.at[pl.ds(r0, rc)])

            @pl.when(~is_num)
            def _():
                pltpu.sync_copy(ceb, den_out.at[pl.ds(r0, rc)])

    return pl.kernel(body, out_type=tuple(out_type), mesh=mesh,
                     scratch_types=scratch)


def kernel(node_id, edge_index, edge_type, h_emb, e_emb,
           A_w, A_b, B_w, B_b, C_w, C_b, D_w, D_b, E_w, E_b):
    N, hid = h_emb.shape
    E = edge_index.shape[1]
    L = A_w.shape[0]
    eb = 80
    n_pad = ((N + NS * eb - 1) // (NS * eb)) * (NS * eb)
    bn = 1000
    be = 2000

    src = edge_index[0]
    dst = edge_index[1]

    w128 = _full((hid, hid))
    b128 = _full((1, hid))

    def bias(b, l):
        return b[l].reshape(1, hid)

    nsd = jax.ShapeDtypeStruct((N, hid), jnp.float32)

    n_et = e_emb.shape[0]
    rep_shape = (NC * NS * n_et, hid)
    tc_pre0 = pl.pallas_call(
        _tc_pre0_body,
        grid=(N // bn,),
        in_specs=[_rows(bn, hid), _full(e_emb.shape)] + [w128, b128] * 5,
        out_specs=[_rows(bn, hid)] * 4 + [_full(rep_shape)] * 2,
        out_shape=[nsd] * 4 + [jax.ShapeDtypeStruct(rep_shape, jnp.float32)] * 2,
    )

    tc_step = pl.pallas_call(
        _tc_step_body,
        grid=(N // bn,),
        in_specs=[_rows(bn, hid)] * 4 + [w128, b128] * 4,
        out_specs=[_rows(bn, hid)] * 5,
        out_shape=[nsd] * 5,
    )

    tc_fin = pl.pallas_call(
        _tc_fin_body,
        grid=(N // bn,),
        in_specs=[_rows(bn, hid)] * 4,
        out_specs=_rows(bn, hid),
        out_shape=nsd,
    )

    tc_ce = pl.pallas_call(
        _tc_ce_body,
        grid=(E // be,),
        in_specs=[_rows(be, hid), w128, b128],
        out_specs=_rows(be, hid),
        out_shape=jax.ShapeDtypeStruct((E, hid), jnp.float32),
    )

    sc_edge0 = _make_sc_edge(True, True, n_pad, E, hid, eb)
    sc_edge_mid = _make_sc_edge(False, True, n_pad, E, hid, eb)
    sc_edge_last = _make_sc_edge(False, False, n_pad, E, hid, eb)

    h = h_emb

    Ah, Bh, Dh, Eh, CeT, EemT = tc_pre0(h, e_emb,
                                        A_w[0], bias(A_b, 0),
                                        B_w[0], bias(B_b, 0),
                                        D_w[0], bias(D_b, 0),
                                        E_w[0], bias(E_b, 0),
                                        C_w[0], bias(C_b, 0))
    e_state, num, den = sc_edge0(src, dst, edge_type, Dh, Eh, Bh, CeT, EemT)

    for l in range(1, L):
        h, Ah, Bh, Dh, Eh = tc_step(h, Ah, num, den,
                                    A_w[l], bias(A_b, l), B_w[l], bias(B_b, l),
                                    D_w[l], bias(D_b, l), E_w[l], bias(E_b, l))
        Ce = tc_ce(e_state, C_w[l], bias(C_b, l))
        if l < L - 1:
            e_state, num, den = sc_edge_mid(src, dst, Dh, Eh, Bh, Ce, e_state)
        else:
            num, den = sc_edge_last(src, dst, Dh, Eh, Bh, Ce)

    return tc_fin(h, Ah, num, den)


# speedup vs baseline: 2.3520x; 1.0144x over previous
import jax
import jax.numpy as jnp
from jax import lax
from jax.experimental import pallas as pl
from jax.experimental.pallas import tpu as pltpu
from jax.experimental.pallas import tpu_sc as plsc

NC = 2
NS = 16
LANE = 16
EPS = 1e-6


def _dot(x, w):
    return jnp.dot(x, w, preferred_element_type=jnp.float32)


def _tc_pre0_body(h_ref, eemb_ref, wa, ba, wb, bb, wd, bd, we, be, wc, bc,
                  ha_o, hb_o, hd_o, he_o, cet_o, eem_o):
    x = h_ref[...]
    ha_o[...] = _dot(x, wa[...]) + ba[...]
    hb_o[...] = _dot(x, wb[...]) + bb[...]
    hd_o[...] = _dot(x, wd[...]) + bd[...]
    he_o[...] = _dot(x, we[...]) + be[...]

    @pl.when(pl.program_id(0) == 0)
    def _():
        em = eemb_ref[...]
        ce = _dot(em, wc[...]) + bc[...]
        n = em.shape[0]
        for k in range(NC * NS):
            cet_o[pl.ds(k * n, n), :] = ce
            eem_o[pl.ds(k * n, n), :] = em


def _tc_step_body(h_ref, hA_ref, num_ref, den_ref,
                  wa, ba, wb, bb, wd, bd, we, be,
                  h_o, ha_o, hb_o, hd_o, he_o):
    h = h_ref[...] + jnp.maximum(
        hA_ref[...] + num_ref[...] / (den_ref[...] + EPS), 0.0)
    h_o[...] = h
    ha_o[...] = _dot(h, wa[...]) + ba[...]
    hb_o[...] = _dot(h, wb[...]) + bb[...]
    hd_o[...] = _dot(h, wd[...]) + bd[...]
    he_o[...] = _dot(h, we[...]) + be[...]


def _tc_fin_body(h_ref, hA_ref, num_ref, den_ref, h_o):
    h_o[...] = h_ref[...] + jnp.maximum(
        hA_ref[...] + num_ref[...] / (den_ref[...] + EPS), 0.0)


def _tc_ce_body(e_ref, wc, bc, o_ref):
    o_ref[...] = _dot(e_ref[...], wc[...]) + bc[...]


def _full(block):
    return pl.BlockSpec(block, lambda i: tuple(0 for _ in block))


def _rows(bn, w):
    return pl.BlockSpec((bn, w), lambda i: (i, 0))


def _make_sc_edge(mode_table, write_e, n_pad, E, hid, eb):
    per_tile = E // NS
    n_blocks = per_tile // eb
    rows_per_tile = n_pad // NS
    rc = eb
    n_chunks = rows_per_tile // rc

    mesh = plsc.VectorSubcoreMesh(core_axis_name="c", subcore_axis_name="s",
                                  num_cores=NC, num_subcores=NS)

    out_type = []
    if write_e:
        out_type.append(jax.ShapeDtypeStruct((E, hid), jnp.float32))
    out_type.append(jax.ShapeDtypeStruct((n_pad, hid), jnp.float32))
    out_type.append(jax.ShapeDtypeStruct((n_pad, hid), jnp.float32))

    scratch = [
        pltpu.VMEM((eb,), jnp.int32),
        pltpu.VMEM((eb,), jnp.int32),
        pltpu.VMEM((eb,), jnp.int32),
        pltpu.VMEM((eb, hid), jnp.float32),
        pltpu.VMEM((eb, hid), jnp.float32),
        pltpu.VMEM((eb, hid), jnp.float32),
        pltpu.VMEM((eb, hid), jnp.float32),
        pltpu.VMEM_SHARED((n_pad, hid), jnp.float32),
        pltpu.SemaphoreType.DMA,
        pltpu.SemaphoreType.DMA,
    ]

    def body(*refs):
        it = iter(refs)
        src_h = next(it)
        dst_h = next(it)
        if mode_table:
            et_h = next(it)
        dh = next(it)
        eh = next(it)
        bh = next(it)
        if mode_table:
            cet = next(it)
            eintab = next(it)
        else:
            ce2 = next(it)
            if write_e:
                ein2 = next(it)
        if write_e:
            eout = next(it)
        num_out = next(it)
        den_out = next(it)
        (srcb, dstb, ietb, gD, gE, aux, ceb, acc, sem, sem_e) = it

        c = lax.axis_index("c")
        s = lax.axis_index("s")
        is_num = c == 0

        zero16 = jnp.zeros((LANE,), jnp.float32)

        def zrow(i, carry):
            for j in range(hid // LANE):
                ceb[i, pl.ds(j * LANE, LANE)] = zero16
            return carry

        lax.fori_loop(0, rc, zrow, 0)
        for k in range(n_chunks):
            r0 = s * rows_per_tile + k * rc
            pltpu.sync_copy(ceb, acc.at[pl.ds(r0, rc)])
        plsc.subcore_barrier()

        tile_e0 = s * per_tile

        def blk(b, carry):
            base = tile_e0 + b * eb
            if write_e:
                @pl.when((~is_num) & (b > 0))
                def _():
                    pltpu.make_async_copy(dh.at[pl.ds(0, eb)], aux,
                                          sem_e).wait()
            pltpu.sync_copy(src_h.at[pl.ds(base, eb)], srcb)
            pltpu.sync_copy(dst_h.at[pl.ds(base, eb)], dstb)
            if mode_table:
                pltpu.sync_copy(et_h.at[pl.ds(base, eb)], ietb)
                rep = (c * NS + s) * 16
                for k in range(eb // LANE):
                    sl = pl.ds(k * LANE, LANE)
                    ietb[sl] = ietb[sl] + rep
            cps = [pltpu.async_copy(dh.at[srcb], gD, sem),
                   pltpu.async_copy(eh.at[dstb], gE, sem)]
            if mode_table:
                cps.append(pltpu.async_copy(cet.at[ietb], ceb, sem))
            else:
                cps.append(pltpu.async_copy(ce2.at[pl.ds(base, eb)], ceb, sem))
            for cp in cps:
                cp.wait()

            @pl.when(is_num)
            def _():
                pltpu.async_copy(bh.at[srcb], aux, sem).wait()

            if write_e:
                @pl.when(~is_num)
                def _():
                    if mode_table:
                        pltpu.async_copy(eintab.at[ietb], aux, sem).wait()
                    else:
                        pltpu.async_copy(ein2.at[pl.ds(base, eb)], aux,
                                         sem).wait()

            def row_num(i, carry2):
                for j in range(hid // LANE):
                    sl = pl.ds(j * LANE, LANE)
                    x = gD[i, sl] + gE[i, sl] + ceb[i, sl]
                    sg = 1.0 / (1.0 + jnp.exp(-x))
                    gD[i, sl] = sg * aux[i, sl]
                return carry2

            def row_den(i, carry2):
                for j in range(hid // LANE):
                    sl = pl.ds(j * LANE, LANE)
                    x = gD[i, sl] + gE[i, sl] + ceb[i, sl]
                    sg = 1.0 / (1.0 + jnp.exp(-x))
                    gD[i, sl] = sg
                    if write_e:
                        aux[i, sl] = aux[i, sl] + jnp.maximum(x, 0.0)
                return carry2

            @pl.when(is_num)
            def _():
                lax.fori_loop(0, eb, row_num, 0)

            @pl.when(~is_num)
            def _():
                lax.fori_loop(0, eb, row_den, 0)

            pltpu.sync_copy(gD, acc.at[dstb], add=True)
            if write_e:
                @pl.when(~is_num)
                def _():
                    pltpu.async_copy(aux, eout.at[pl.ds(base, eb)], sem_e)
            return carry

        lax.fori_loop(0, n_blocks, blk, 0)
        if write_e:
            @pl.when(~is_num)
            def _():
                pltpu.make_async_copy(dh.at[pl.ds(0, eb)], aux, sem_e).wait()
        plsc.subcore_barrier()

        for k in range(n_chunks):
            r0 = s * rows_per_tile + k * rc
            pltpu.sync_copy(acc.at[pl.ds(r0, rc)], ceb)

            @pl.when(is_num)
            def _():
                pltpu.sync_copy(ceb, num_out.at[pl.ds(r0, rc)])

            @pl.when(~is_num)
            def _():
                pltpu.sync_copy(ceb, den_out.at[pl.ds(r0, rc)])

    return pl.kernel(body, out_type=tuple(out_type), mesh=mesh,
                     scratch_types=scratch)


def kernel(node_id, edge_index, edge_type, h_emb, e_emb,
           A_w, A_b, B_w, B_b, C_w, C_b, D_w, D_b, E_w, E_b):
    N, hid = h_emb.shape
    E = edge_index.shape[1]
    L = A_w.shape[0]
    eb = 80
    n_pad = ((N + NS * eb - 1) // (NS * eb)) * (NS * eb)
    bn = 1000
    be = 2000

    src = edge_index[0]
    dst = edge_index[1]

    w128 = _full((hid, hid))
    b128 = _full((1, hid))

    def bias(b, l):
        return b[l].reshape(1, hid)

    nsd = jax.ShapeDtypeStruct((N, hid), jnp.float32)

    n_et = e_emb.shape[0]
    rep_shape = (NC * NS * n_et, hid)
    tc_pre0 = pl.pallas_call(
        _tc_pre0_body,
        grid=(N // bn,),
        in_specs=[_rows(bn, hid), _full(e_emb.shape)] + [w128, b128] * 5,
        out_specs=[_rows(bn, hid)] * 4 + [_full(rep_shape)] * 2,
        out_shape=[nsd] * 4 + [jax.ShapeDtypeStruct(rep_shape, jnp.float32)] * 2,
    )

    tc_step = pl.pallas_call(
        _tc_step_body,
        grid=(N // bn,),
        in_specs=[_rows(bn, hid)] * 4 + [w128, b128] * 4,
        out_specs=[_rows(bn, hid)] * 5,
        out_shape=[nsd] * 5,
    )

    tc_fin = pl.pallas_call(
        _tc_fin_body,
        grid=(N // bn,),
        in_specs=[_rows(bn, hid)] * 4,
        out_specs=_rows(bn, hid),
        out_shape=nsd,
    )

    tc_ce = pl.pallas_call(
        _tc_ce_body,
        grid=(E // be,),
        in_specs=[_rows(be, hid), w128, b128],
        out_specs=_rows(be, hid),
        out_shape=jax.ShapeDtypeStruct((E, hid), jnp.float32),
    )

    sc_edge0 = _make_sc_edge(True, True, n_pad, E, hid, eb)
    sc_edge_mid = _make_sc_edge(False, True, n_pad, E, hid, eb)
    sc_edge_last = _make_sc_edge(False, False, n_pad, E, hid, eb)

    h = h_emb

    Ah, Bh, Dh, Eh, CeT, EemT = tc_pre0(h, e_emb,
                                        A_w[0], bias(A_b, 0),
                                        B_w[0], bias(B_b, 0),
                                        D_w[0], bias(D_b, 0),
                                        E_w[0], bias(E_b, 0),
                                        C_w[0], bias(C_b, 0))
    e_state, num, den = sc_edge0(src, dst, edge_type, Dh, Eh, Bh, CeT, EemT)

    for l in range(1, L):
        h, Ah, Bh, Dh, Eh = tc_step(h, Ah, num, den,
                                    A_w[l], bias(A_b, l), B_w[l], bias(B_b, l),
                                    D_w[l], bias(D_b, l), E_w[l], bias(E_b, l))
        Ce = tc_ce(e_state, C_w[l], bias(C_b, l))
        if l < L - 1:
            e_state, num, den = sc_edge_mid(src, dst, Dh, Eh, Bh, Ce, e_state)
        else:
            num, den = sc_edge_last(src, dst, Dh, Eh, Bh, Ce)

    return tc_fin(h, Ah, num, den)
